# Initial kernel scaffold; baseline (speedup 1.0000x reference)
import jax
import jax.numpy as jnp
from jax.experimental import pallas as pl


def kernel(features, nodes, neighbours_full, num_sample):
    raise NotImplementedError("write your pallas kernel here")


# speedup vs baseline: 6.6839x; 6.6839x over previous
import functools

import jax
import jax.numpy as jnp
import numpy as np
from jax import lax
from jax.experimental import pallas as pl
from jax.experimental.pallas import tpu as pltpu
from jax.experimental.pallas import tpu_sc as plsc

_NUM_CORES = 2
_NUM_SUBCORES = 16
_NUM_WORKERS = _NUM_CORES * _NUM_SUBCORES

_B = 50000
_D = 128
_S1 = 11
_C = 56
_NCHUNK = 896
_CHUNKS_PER_W = _NCHUNK // _NUM_WORKERS
_INV = 1.0 / _S1


def _sc_body(feat_hbm, idxc_hbm, out_hbm, idx_v, gbuf, sem):
    wid = lax.axis_index("c") * _NUM_SUBCORES + lax.axis_index("s")

    def chunk_body(t, _):
        i = wid + t * _NUM_WORKERS
        row0 = jnp.minimum(i * _C, _B - _C)
        pltpu.sync_copy(idxc_hbm.at[i], idx_v)
        descs = [
            pltpu.async_copy(feat_hbm.at[idx_v.at[k]], gbuf.at[k], sem)
            for k in range(_S1)
        ]
        for d in descs:
            d.wait()

        def reduce_row(r, _):
            for j in range(_D // 16):
                sl = pl.ds(j * 16, 16)
                acc = gbuf[0, r, sl]
                for k in range(1, _S1):
                    acc = acc + gbuf[k, r, sl]
                gbuf[0, r, sl] = acc * _INV
            return _

        lax.fori_loop(0, _C, reduce_row, None)
        pltpu.sync_copy(gbuf.at[0], out_hbm.at[pl.ds(row0, _C)])
        return _

    lax.fori_loop(0, _CHUNKS_PER_W, chunk_body, None)


@functools.partial(
    pl.kernel,
    out_type=jax.ShapeDtypeStruct((_B, _D), jnp.float32),
    mesh=plsc.VectorSubcoreMesh(
        core_axis_name="c", subcore_axis_name="s",
        num_cores=_NUM_CORES, num_subcores=_NUM_SUBCORES,
    ),
    scratch_types=[
        pltpu.VMEM((_S1, _C), jnp.int32),
        pltpu.VMEM((_S1, _C, _D), jnp.float32),
        pltpu.SemaphoreType.DMA,
    ],
)
def _mean_agg_sc(feat_hbm, idxc_hbm, out_hbm, idx_v, gbuf, sem):
    _sc_body(feat_hbm, idxc_hbm, out_hbm, idx_v, gbuf, sem)


def kernel(features, nodes, neighbours_full, num_sample):
    s = neighbours_full.shape[1]
    all_idx = jnp.concatenate([neighbours_full, nodes[:, None]], axis=1)
    all_idx = all_idx + (num_sample - s)
    starts = np.minimum(np.arange(_NCHUNK) * _C, _B - _C)
    rows = (starts[:, None] + np.arange(_C)[None, :]).astype(np.int32)
    idxc = jnp.take(all_idx, jnp.asarray(rows.reshape(-1)), axis=0)
    idxc = idxc.reshape(_NCHUNK, _C, _S1).transpose(0, 2, 1)
    return _mean_agg_sc(features, idxc)


# speedup vs baseline: 9.5892x; 1.4347x over previous
import functools

import jax
import jax.numpy as jnp
import numpy as np
from jax import lax
from jax.experimental import pallas as pl
from jax.experimental.pallas import tpu as pltpu
from jax.experimental.pallas import tpu_sc as plsc

_NUM_CORES = 2
_NUM_SUBCORES = 16
_NUM_WORKERS = _NUM_CORES * _NUM_SUBCORES

_B = 50000
_D = 128
_S1 = 11
_C = 32
_NCHUNK = 1568
_CPW = _NCHUNK // _NUM_WORKERS
_INV = 1.0 / _S1


def _sc_body(feat_hbm, idxc_hbm, out_hbm, idx_all, gbuf, sem0, sem1):
    wid = lax.axis_index("c") * _NUM_SUBCORES + lax.axis_index("s")
    sems = (sem0, sem1)

    pltpu.sync_copy(idxc_hbm.at[wid], idx_all)

    def fire(t, b):
        for k in range(_S1):
            idx = idx_all.at[pl.ds((t * _S1 + k) * _C, _C)]
            pltpu.async_copy(feat_hbm.at[idx], gbuf.at[b, k], sems[b])

    def drain(b):
        for k in range(_S1):
            pltpu.make_async_copy(feat_hbm.at[pl.ds(0, _C)], gbuf.at[b, k],
                                  sems[b]).wait()

    def reduce_store(b, t):
        row0 = jnp.minimum((wid * _CPW + t) * _C, _B - _C)

        def reduce_row(r, _):
            for j in range(_D // 16):
                sl = pl.ds(j * 16, 16)
                acc = gbuf[b, 0, r, sl]
                for k in range(1, _S1):
                    acc = acc + gbuf[b, k, r, sl]
                gbuf[b, 0, r, sl] = acc * _INV
            return _

        lax.fori_loop(0, _C, reduce_row, None)
        pltpu.sync_copy(gbuf.at[b, 0], out_hbm.at[pl.ds(row0, _C)])

    fire(0, 0)

    def pair_body(t2, _):
        t = 2 * t2
        fire(t + 1, 1)
        drain(0)
        reduce_store(0, t)
        fire(t + 2, 0)
        drain(1)
        reduce_store(1, t + 1)
        return _

    lax.fori_loop(0, (_CPW - 1) // 2, pair_body, None)
    drain(0)
    reduce_store(0, _CPW - 1)


@functools.partial(
    pl.kernel,
    out_type=jax.ShapeDtypeStruct((_B, _D), jnp.float32),
    mesh=plsc.VectorSubcoreMesh(
        core_axis_name="c", subcore_axis_name="s",
        num_cores=_NUM_CORES, num_subcores=_NUM_SUBCORES,
    ),
    scratch_types=[
        pltpu.VMEM((_CPW * _S1 * _C,), jnp.int32),
        pltpu.VMEM((2, _S1, _C, _D), jnp.float32),
        pltpu.SemaphoreType.DMA,
        pltpu.SemaphoreType.DMA,
    ],
)
def _mean_agg_sc(feat_hbm, idxc_hbm, out_hbm, idx_all, gbuf, sem0, sem1):
    _sc_body(feat_hbm, idxc_hbm, out_hbm, idx_all, gbuf, sem0, sem1)


def kernel(features, nodes, neighbours_full, num_sample):
    s = neighbours_full.shape[1]
    all_idx = jnp.concatenate([neighbours_full, nodes[:, None]], axis=1)
    all_idx = all_idx + (num_sample - s)
    starts = np.minimum(np.arange(_NCHUNK) * _C, _B - _C)
    rows = (starts[:, None] + np.arange(_C)[None, :]).astype(np.int32)
    idxc = jnp.take(all_idx, jnp.asarray(rows.reshape(-1)), axis=0)
    idxc = idxc.reshape(_NCHUNK, _C, _S1).transpose(0, 2, 1)
    idxc = idxc.reshape(_NUM_WORKERS, _CPW * _S1 * _C)
    return _mean_agg_sc(features, idxc)
